# speedup vs baseline: 8.2211x; 8.2211x over previous
import functools

import jax
import jax.numpy as jnp
from jax import lax
from jax.experimental import pallas as pl
from jax.experimental.pallas import tpu as pltpu, tpu_sc as plsc

VOCAB = 100000
D_MODEL = 128
BATCH = 4096
HIST = 200

_B = BATCH * HIST
_NC, _NS = 2, 16
_NW = _NC * _NS
_BPW = _B // _NW
_CHUNK = 512
_NCHUNK = _BPW // _CHUNK

_mesh = plsc.VectorSubcoreMesh(core_axis_name="c", subcore_axis_name="s")


@functools.partial(
    pl.kernel,
    out_type=jax.ShapeDtypeStruct((_B, D_MODEL), jnp.float32),
    mesh=_mesh,
    scratch_types=[
        pltpu.VMEM((_CHUNK,), jnp.int32),
        pltpu.VMEM((_CHUNK, D_MODEL), jnp.float32),
        pltpu.SemaphoreType.DMA,
    ],
)
def _gather_kernel(idx_hbm, table_hbm, out_hbm, idx_v, rows_v, sem):
    wid = lax.axis_index("s") * _NC + lax.axis_index("c")
    base = wid * _BPW

    def step(i, carry):
        off = base + i * _CHUNK
        pltpu.sync_copy(idx_hbm.at[pl.ds(off, _CHUNK)], idx_v)
        pltpu.async_copy(table_hbm.at[idx_v], rows_v, sem).wait()
        pltpu.sync_copy(rows_v, out_hbm.at[pl.ds(off, _CHUNK)])
        return carry

    lax.fori_loop(0, _NCHUNK, step, 0)


def kernel(x, table):
    idx = x.reshape(_B).astype(jnp.int32)
    out = _gather_kernel(idx, table)
    return out.reshape(BATCH, HIST, D_MODEL)


# speedup vs baseline: 9.3434x; 1.1365x over previous
import functools

import jax
import jax.numpy as jnp
from jax import lax
from jax.experimental import pallas as pl
from jax.experimental.pallas import tpu as pltpu, tpu_sc as plsc

VOCAB = 100000
D_MODEL = 128
BATCH = 4096
HIST = 200

_B = BATCH * HIST
_NC, _NS = 2, 16
_NW = _NC * _NS
_BPW = _B // _NW
_NBUF = 2
_CHUNK = 400
_NCHUNK = _BPW // _CHUNK
_NGRP = _NCHUNK // _NBUF

_mesh = plsc.VectorSubcoreMesh(core_axis_name="c", subcore_axis_name="s")


@functools.partial(
    pl.kernel,
    out_type=jax.ShapeDtypeStruct((_B, D_MODEL), jnp.float32),
    mesh=_mesh,
    scratch_types=[
        pltpu.VMEM((_CHUNK,), jnp.int32),
        pltpu.VMEM((_CHUNK,), jnp.int32),
        pltpu.VMEM((_CHUNK, D_MODEL), jnp.float32),
        pltpu.VMEM((_CHUNK, D_MODEL), jnp.float32),
        pltpu.SemaphoreType.DMA,
        pltpu.SemaphoreType.DMA,
        pltpu.SemaphoreType.DMA,
        pltpu.SemaphoreType.DMA,
    ],
)
def _gather_kernel(idx_hbm, table_hbm, out_hbm,
                   idx0, idx1, rows0, rows1, g0, g1, s0, s1):
    idx_v = (idx0, idx1)
    rows_v = (rows0, rows1)
    gsem = (g0, g1)
    ssem = (s0, s1)

    wid = lax.axis_index("s") * _NC + lax.axis_index("c")
    base = wid * _BPW

    for b in range(_NBUF):
        off = base + b * _CHUNK
        pltpu.sync_copy(idx_hbm.at[pl.ds(off, _CHUNK)], idx_v[b])
        pltpu.async_copy(table_hbm.at[idx_v[b]], rows_v[b], gsem[b])

    def group(g, carry):
        for b in range(_NBUF):
            i = g * _NBUF + b
            off = base + i * _CHUNK
            noff = off + _NBUF * _CHUNK
            pltpu.make_async_copy(
                table_hbm.at[idx_v[b]], rows_v[b], gsem[b]).wait()
            pltpu.async_copy(rows_v[b], out_hbm.at[pl.ds(off, _CHUNK)],
                             ssem[b])
            pltpu.sync_copy(idx_hbm.at[pl.ds(noff, _CHUNK)], idx_v[b])
            pltpu.make_async_copy(
                rows_v[b], out_hbm.at[pl.ds(off, _CHUNK)], ssem[b]).wait()
            pltpu.async_copy(table_hbm.at[idx_v[b]], rows_v[b], gsem[b])
        return carry

    lax.fori_loop(0, _NGRP - 1, group, 0)

    for b in range(_NBUF):
        i = (_NGRP - 1) * _NBUF + b
        off = base + i * _CHUNK
        pltpu.make_async_copy(
            table_hbm.at[idx_v[b]], rows_v[b], gsem[b]).wait()
        pltpu.async_copy(rows_v[b], out_hbm.at[pl.ds(off, _CHUNK)], ssem[b])
    for b in range(_NBUF):
        i = (_NGRP - 1) * _NBUF + b
        off = base + i * _CHUNK
        pltpu.make_async_copy(
            rows_v[b], out_hbm.at[pl.ds(off, _CHUNK)], ssem[b]).wait()


def kernel(x, table):
    idx = x.reshape(_B).astype(jnp.int32)
    out = _gather_kernel(idx, table)
    return out.reshape(BATCH, HIST, D_MODEL)


# speedup vs baseline: 9.3779x; 1.0037x over previous
import functools

import jax
import jax.numpy as jnp
from jax import lax
from jax.experimental import pallas as pl
from jax.experimental.pallas import tpu as pltpu, tpu_sc as plsc

VOCAB = 100000
D_MODEL = 128
BATCH = 4096
HIST = 200

_B = BATCH * HIST
_NC, _NS = 2, 16
_NW = _NC * _NS
_BPW = _B // _NW
_NBUF = 4
_CHUNK = 200
_NCHUNK = _BPW // _CHUNK
_NGRP = _NCHUNK // _NBUF

_mesh = plsc.VectorSubcoreMesh(core_axis_name="c", subcore_axis_name="s")


@functools.partial(
    pl.kernel,
    out_type=jax.ShapeDtypeStruct((_B, D_MODEL), jnp.float32),
    mesh=_mesh,
    scratch_types=[
        pltpu.VMEM((_BPW,), jnp.int32),
        *[pltpu.VMEM((_CHUNK, D_MODEL), jnp.float32) for _ in range(_NBUF)],
        *[pltpu.SemaphoreType.DMA for _ in range(2 * _NBUF)],
    ],
)
def _gather_kernel(idx_hbm, table_hbm, out_hbm, idx_v, *bufs):
    rows_v = bufs[:_NBUF]
    gsem = bufs[_NBUF:2 * _NBUF]
    ssem = bufs[2 * _NBUF:]

    wid = lax.axis_index("s") * _NC + lax.axis_index("c")
    base = wid * _BPW

    pltpu.sync_copy(idx_hbm.at[pl.ds(base, _BPW)], idx_v)

    def chunk_idx(i):
        return idx_v.at[pl.ds(i * _CHUNK, _CHUNK)]

    for b in range(_NBUF):
        pltpu.async_copy(table_hbm.at[chunk_idx(b)], rows_v[b], gsem[b])

    def group(g, carry):
        for b in range(_NBUF):
            i = g * _NBUF + b
            off = base + i * _CHUNK
            pltpu.make_async_copy(
                table_hbm.at[chunk_idx(i)], rows_v[b], gsem[b]).wait()
            pltpu.async_copy(rows_v[b], out_hbm.at[pl.ds(off, _CHUNK)],
                             ssem[b])
            pltpu.make_async_copy(
                rows_v[b], out_hbm.at[pl.ds(off, _CHUNK)], ssem[b]).wait()
            pltpu.async_copy(table_hbm.at[chunk_idx(i + _NBUF)],
                             rows_v[b], gsem[b])
        return carry

    lax.fori_loop(0, _NGRP - 1, group, 0)

    for b in range(_NBUF):
        i = (_NGRP - 1) * _NBUF + b
        off = base + i * _CHUNK
        pltpu.make_async_copy(
            table_hbm.at[chunk_idx(i)], rows_v[b], gsem[b]).wait()
        pltpu.async_copy(rows_v[b], out_hbm.at[pl.ds(off, _CHUNK)], ssem[b])
    for b in range(_NBUF):
        i = (_NGRP - 1) * _NBUF + b
        off = base + i * _CHUNK
        pltpu.make_async_copy(
            rows_v[b], out_hbm.at[pl.ds(off, _CHUNK)], ssem[b]).wait()


def kernel(x, table):
    idx = x.reshape(_B).astype(jnp.int32)
    out = _gather_kernel(idx, table)
    return out.reshape(BATCH, HIST, D_MODEL)
